# baseline (speedup 1.0000x reference)
import jax
import jax.numpy as jnp
from jax import lax
from jax.experimental import pallas as pl
from jax.experimental.pallas import tpu as pltpu
from jax.experimental.pallas import tpu_sc as plsc

N_CORES = 2
N_SUBCORES = 16
N_WORKERS = N_CORES * N_SUBCORES


def _sc_bcast(w_hbm, out_hbm, sem):
    batch = out_hbm.shape[0]
    seq_len = out_hbm.shape[1]
    wid = lax.axis_index("s") * N_CORES + lax.axis_index("c")
    rows_per_w = seq_len // N_WORKERS
    base = wid * rows_per_w
    src = w_hbm.at[pl.ds(base, rows_per_w)]
    for b in range(batch):
        pltpu.async_copy(src, out_hbm.at[b, pl.ds(base, rows_per_w)], sem)
    for b in range(batch):
        pltpu.make_async_copy(src, out_hbm.at[b, pl.ds(base, rows_per_w)], sem).wait()


def kernel(tokens, W_pos):
    batch, seq_len = tokens.shape
    d = W_pos.shape[1]
    mesh = plsc.VectorSubcoreMesh(core_axis_name="c", subcore_axis_name="s")
    k = pl.kernel(
        _sc_bcast,
        mesh=mesh,
        out_type=jax.ShapeDtypeStruct((batch, seq_len, d), W_pos.dtype),
        scratch_types=[pltpu.SemaphoreType.DMA],
    )
    return k(W_pos[:seq_len])


# speedup vs baseline: 54.2029x; 54.2029x over previous
import jax
import jax.numpy as jnp
from jax import lax
from jax.experimental import pallas as pl
from jax.experimental.pallas import tpu as pltpu
from jax.experimental.pallas import tpu_sc as plsc

N_CORES = 2
N_SUBCORES = 16
N_WORKERS = N_CORES * N_SUBCORES
CHUNK_ROWS = 32


def _sc_bcast(w_hbm, out_hbm, buf0, buf1, in_sem, out_sem):
    batch = out_hbm.shape[0]
    seq_len = out_hbm.shape[1]
    wid = lax.axis_index("s") * N_CORES + lax.axis_index("c")
    rows_per_w = seq_len // N_WORKERS
    n_chunks = rows_per_w // CHUNK_ROWS
    base0 = wid * rows_per_w
    bufs = [buf0, buf1]

    def src(c):
        return w_hbm.at[pl.ds(base0 + c * CHUNK_ROWS, CHUNK_ROWS)]

    def dst(c, b):
        return out_hbm.at[b, pl.ds(base0 + c * CHUNK_ROWS, CHUNK_ROWS)]

    pltpu.async_copy(src(0), bufs[0], in_sem)
    for c in range(n_chunks):
        cur = bufs[c % 2]
        nxt = bufs[(c + 1) % 2]
        pltpu.make_async_copy(src(c), cur, in_sem).wait()
        if c >= 1:
            for b in range(batch):
                pltpu.make_async_copy(nxt, dst(c - 1, b), out_sem).wait()
        if c + 1 < n_chunks:
            pltpu.async_copy(src(c + 1), nxt, in_sem)
        for b in range(batch):
            pltpu.async_copy(cur, dst(c, b), out_sem)
    last = bufs[(n_chunks - 1) % 2]
    for b in range(batch):
        pltpu.make_async_copy(last, dst(n_chunks - 1, b), out_sem).wait()


def kernel(tokens, W_pos):
    batch, seq_len = tokens.shape
    d = W_pos.shape[1]
    mesh = plsc.VectorSubcoreMesh(core_axis_name="c", subcore_axis_name="s")
    k = pl.kernel(
        _sc_bcast,
        mesh=mesh,
        out_type=jax.ShapeDtypeStruct((batch, seq_len, d), W_pos.dtype),
        scratch_types=[
            pltpu.VMEM((CHUNK_ROWS, d), W_pos.dtype),
            pltpu.VMEM((CHUNK_ROWS, d), W_pos.dtype),
            pltpu.SemaphoreType.DMA,
            pltpu.SemaphoreType.DMA,
        ],
    )
    return k(W_pos[:seq_len])


# speedup vs baseline: 75.2626x; 1.3885x over previous
import jax
import jax.numpy as jnp
from jax.experimental import pallas as pl
from jax.experimental.pallas import tpu as pltpu


def _make_body(bs, batch):
    def body(w_ref, out_ref, sem):
        i = pl.program_id(0)
        for b in range(batch):
            pltpu.make_async_copy(
                w_ref, out_ref.at[b, pl.ds(i * bs, bs)], sem
            ).start()
        for b in range(batch):
            pltpu.make_async_copy(
                w_ref, out_ref.at[b, pl.ds(i * bs, bs)], sem
            ).wait()

    return body


def kernel(tokens, W_pos):
    batch, seq_len = tokens.shape
    d = W_pos.shape[1]
    bs = 1024
    grid = (seq_len // bs,)
    out = pl.pallas_call(
        _make_body(bs, batch),
        grid=grid,
        in_specs=[pl.BlockSpec((bs, d), lambda i: (i, 0))],
        out_specs=pl.BlockSpec(memory_space=pl.ANY),
        out_shape=jax.ShapeDtypeStruct((batch, seq_len, d), W_pos.dtype),
        scratch_shapes=[pltpu.SemaphoreType.DMA],
    )(W_pos[:seq_len])
    return out


# speedup vs baseline: 80.7895x; 1.0734x over previous
import jax
import jax.numpy as jnp
from jax.experimental import pallas as pl
from jax.experimental.pallas import tpu as pltpu


def _make_body(bs, batch):
    def body(w_ref, out_ref, buf0, buf1, sem0, sem1):
        i = pl.program_id(0)
        n = pl.num_programs(0)
        bufs = [buf0, buf1]
        sems = [sem0, sem1]

        def fan_copies(buf, sem, step):
            return [
                pltpu.make_async_copy(
                    buf, out_ref.at[b, pl.ds(step * bs, bs)], sem
                )
                for b in range(batch)
            ]

        @pl.when(i % 2 == 0)
        def _even():
            @pl.when(i >= 2)
            def _drain():
                for c in fan_copies(bufs[0], sems[0], i - 2):
                    c.wait()

            bufs[0][...] = w_ref[...]
            for c in fan_copies(bufs[0], sems[0], i):
                c.start()

        @pl.when(i % 2 == 1)
        def _odd():
            @pl.when(i >= 2)
            def _drain():
                for c in fan_copies(bufs[1], sems[1], i - 2):
                    c.wait()

            bufs[1][...] = w_ref[...]
            for c in fan_copies(bufs[1], sems[1], i):
                c.start()

        @pl.when(i == n - 1)
        def _final_drain():
            for c in fan_copies(bufs[(n - 2) % 2], sems[(n - 2) % 2], n - 2):
                c.wait()
            for c in fan_copies(bufs[(n - 1) % 2], sems[(n - 1) % 2], n - 1):
                c.wait()

    return body


def kernel(tokens, W_pos):
    batch, seq_len = tokens.shape
    d = W_pos.shape[1]
    bs = 1024
    grid = (seq_len // bs,)
    out = pl.pallas_call(
        _make_body(bs, batch),
        grid=grid,
        in_specs=[pl.BlockSpec((bs, d), lambda i: (i, 0))],
        out_specs=pl.BlockSpec(memory_space=pl.ANY),
        out_shape=jax.ShapeDtypeStruct((batch, seq_len, d), W_pos.dtype),
        scratch_shapes=[
            pltpu.VMEM((bs, d), W_pos.dtype),
            pltpu.VMEM((bs, d), W_pos.dtype),
            pltpu.SemaphoreType.DMA,
            pltpu.SemaphoreType.DMA,
        ],
    )(W_pos[:seq_len])
    return out


# speedup vs baseline: 82.9693x; 1.0270x over previous
import jax
import jax.numpy as jnp
from jax.experimental import pallas as pl
from jax.experimental.pallas import tpu as pltpu


def _make_body(bs, batch):
    def body(w_ref, out_ref, buf0, buf1, sem0, sem1):
        i = pl.program_id(0)
        n = pl.num_programs(0)
        bufs = [buf0, buf1]
        sems = [sem0, sem1]

        def fan_copies(buf, sem, step):
            return [
                pltpu.make_async_copy(
                    buf, out_ref.at[b, pl.ds(step * bs, bs)], sem
                )
                for b in range(batch)
            ]

        @pl.when(i % 2 == 0)
        def _even():
            @pl.when(i >= 2)
            def _drain():
                for c in fan_copies(bufs[0], sems[0], i - 2):
                    c.wait()

            bufs[0][...] = w_ref[...]
            for c in fan_copies(bufs[0], sems[0], i):
                c.start()

        @pl.when(i % 2 == 1)
        def _odd():
            @pl.when(i >= 2)
            def _drain():
                for c in fan_copies(bufs[1], sems[1], i - 2):
                    c.wait()

            bufs[1][...] = w_ref[...]
            for c in fan_copies(bufs[1], sems[1], i):
                c.start()

        @pl.when(i == n - 1)
        def _final_drain():
            for c in fan_copies(bufs[(n - 2) % 2], sems[(n - 2) % 2], n - 2):
                c.wait()
            for c in fan_copies(bufs[(n - 1) % 2], sems[(n - 1) % 2], n - 1):
                c.wait()

    return body


def kernel(tokens, W_pos):
    batch, seq_len = tokens.shape
    d = W_pos.shape[1]
    bs = 2048
    grid = (seq_len // bs,)
    out = pl.pallas_call(
        _make_body(bs, batch),
        grid=grid,
        in_specs=[pl.BlockSpec((bs, d), lambda i: (i, 0))],
        out_specs=pl.BlockSpec(memory_space=pl.ANY),
        out_shape=jax.ShapeDtypeStruct((batch, seq_len, d), W_pos.dtype),
        scratch_shapes=[
            pltpu.VMEM((bs, d), W_pos.dtype),
            pltpu.VMEM((bs, d), W_pos.dtype),
            pltpu.SemaphoreType.DMA,
            pltpu.SemaphoreType.DMA,
        ],
    )(W_pos[:seq_len])
    return out
